# Initial kernel scaffold; baseline (speedup 1.0000x reference)
import jax
import jax.numpy as jnp
from jax.experimental import pallas as pl


def kernel(x, edge_index, feat_gate, edge_gate, W_proj, b_proj, W1, b1, W2, b2):
    raise NotImplementedError("write your pallas kernel here")


# speedup vs baseline: 11.0483x; 11.0483x over previous
import functools
import jax
import jax.numpy as jnp
from jax import lax
from jax.experimental import pallas as pl
from jax.experimental.pallas import tpu as pltpu
from jax.experimental.pallas import tpu_sc as plsc

N = 10000
E = 320000
D = 128
H = 128

NC = 2
NS = 16
NW = NC * NS
CH = 128

N_PAD = 10240
CHUNKS = -(-E // (NW * CH))
E_PAD = CHUNKS * NW * CH
ROWS_PER_TILE = N_PAD // NS

_mesh = plsc.VectorSubcoreMesh(core_axis_name="c", subcore_axis_name="s")


@functools.partial(
    pl.kernel,
    out_type=(
        jax.ShapeDtypeStruct((NC, N_PAD), jnp.float32),
        jax.ShapeDtypeStruct((NC, N_PAD), jnp.float32),
    ),
    mesh=_mesh,
    scratch_types=[
        pltpu.VMEM((1, CH), jnp.int32),
        pltpu.VMEM((1, CH), jnp.int32),
        pltpu.VMEM((1, CH), jnp.float32),
        pltpu.VMEM((1, CH), jnp.float32),
        pltpu.VMEM((ROWS_PER_TILE,), jnp.float32),
        pltpu.VMEM_SHARED((N_PAD,), jnp.float32),
        pltpu.VMEM_SHARED((N_PAD,), jnp.float32),
    ],
)
def _sc_edge_scalars(src_hbm, dst_hbm, w_hbm, deg_out, craw_out,
                     srcv, dstv, wv, wzv, stage, deg_s, craw_s):
    cid = lax.axis_index("c")
    sid = lax.axis_index("s")
    wid = sid * NC + cid

    def zero_body(i, _):
        stage[pl.ds(i * 16, 16)] = jnp.zeros((16,), jnp.float32)
        return 0
    lax.fori_loop(0, ROWS_PER_TILE // 16, zero_body, 0)
    base = sid * ROWS_PER_TILE
    pltpu.sync_copy(stage, deg_s.at[pl.ds(base, ROWS_PER_TILE)])
    pltpu.sync_copy(stage, craw_s.at[pl.ds(base, ROWS_PER_TILE)])
    plsc.subcore_barrier()

    def body(i, _):
        off = (wid * CHUNKS + i) * CH
        pltpu.sync_copy(src_hbm.at[pl.ds(off, CH)], srcv.at[0])
        pltpu.sync_copy(dst_hbm.at[pl.ds(off, CH)], dstv.at[0])
        pltpu.sync_copy(w_hbm.at[pl.ds(off, CH)], wv.at[0])
        for j in range(CH // 16):
            sl = pl.ds(j * 16, 16)
            m = dstv[0, sl] == 0
            wzv[0, sl] = jnp.where(m, wv[0, sl], jnp.zeros((16,), jnp.float32))
        pltpu.sync_copy(wv.at[0], deg_s.at[dstv.at[0]], add=True)
        pltpu.sync_copy(wzv.at[0], craw_s.at[srcv.at[0]], add=True)
        return 0
    lax.fori_loop(0, CHUNKS, body, 0)
    plsc.subcore_barrier()

    pltpu.sync_copy(deg_s.at[pl.ds(base, ROWS_PER_TILE)], stage)
    pltpu.sync_copy(stage, deg_out.at[cid, pl.ds(base, ROWS_PER_TILE)])
    pltpu.sync_copy(craw_s.at[pl.ds(base, ROWS_PER_TILE)], stage)
    pltpu.sync_copy(stage, craw_out.at[cid, pl.ds(base, ROWS_PER_TILE)])


@functools.partial(
    pl.kernel,
    out_type=jax.ShapeDtypeStruct((NC, N_PAD, H), jnp.float32),
    mesh=_mesh,
    scratch_types=[
        pltpu.VMEM((1, CH), jnp.int32),
        pltpu.VMEM((1, CH), jnp.int32),
        pltpu.VMEM((1, CH), jnp.float32),
        pltpu.VMEM((CH, H), jnp.float32),
        pltpu.VMEM_SHARED((N_PAD, H), jnp.float32),
        pltpu.SemaphoreType.DMA,
    ],
)
def _sc_scatter_rows(src_hbm, dst_hbm, w_hbm, tab_hbm, s_out,
                     srcv, dstv, wv, rows, s_s, sem):
    cid = lax.axis_index("c")
    sid = lax.axis_index("s")
    wid = sid * NC + cid

    def zrow(i, _):
        for r in range(H // 16):
            rows[i, pl.ds(r * 16, 16)] = jnp.zeros((16,), jnp.float32)
        return 0
    lax.fori_loop(0, CH, zrow, 0)
    base = sid * ROWS_PER_TILE
    for k in range(ROWS_PER_TILE // CH):
        pltpu.sync_copy(rows, s_s.at[pl.ds(base + k * CH, CH), :])
    plsc.subcore_barrier()

    def body(i, _):
        off = (wid * CHUNKS + i) * CH
        pltpu.sync_copy(src_hbm.at[pl.ds(off, CH)], srcv.at[0])
        pltpu.sync_copy(dst_hbm.at[pl.ds(off, CH)], dstv.at[0])
        pltpu.sync_copy(w_hbm.at[pl.ds(off, CH)], wv.at[0])
        pltpu.async_copy(tab_hbm.at[srcv.at[0]], rows, sem).wait()

        def scale(g, _):
            w16 = wv[0, pl.ds(g * 16, 16)]
            for k in range(16):
                wk = jnp.broadcast_to(w16[k], (16,))
                j = g * 16 + k
                for r in range(H // 16):
                    sl = pl.ds(r * 16, 16)
                    rows[j, sl] = rows[j, sl] * wk
            return 0
        lax.fori_loop(0, CH // 16, scale, 0)
        pltpu.sync_copy(rows, s_s.at[dstv.at[0]], add=True)
        return 0
    lax.fori_loop(0, CHUNKS, body, 0)
    plsc.subcore_barrier()

    for k in range(ROWS_PER_TILE // CH):
        sl = pl.ds(base + k * CH, CH)
        pltpu.sync_copy(s_s.at[sl, :], rows)
        pltpu.sync_copy(rows, s_out.at[cid, sl, :])


def _tc_scalars_body(deg_ref, craw_ref, rs_ref, cw_ref):
    deg = deg_ref[0] + deg_ref[1] + 1.0
    rs = lax.rsqrt(deg)
    rs_ref[...] = rs
    cw_ref[...] = (craw_ref[0] + craw_ref[1]) * rs


def _tc_scalars(deg_p, craw_p):
    r = N_PAD // 128
    return pl.pallas_call(
        _tc_scalars_body,
        out_shape=(
            jax.ShapeDtypeStruct((r, 128), jnp.float32),
            jax.ShapeDtypeStruct((r, 128), jnp.float32),
        ),
    )(deg_p.reshape(NC, r, 128), craw_p.reshape(NC, r, 128))


_BLK = 1000


def _tc_matmul_body(x_ref, g_ref, wp_ref, w1_ref, bp_ref, b1_ref, rs_ref,
                    out_ref, wf_ref, bf_ref):
    @pl.when(pl.program_id(0) == 0)
    def _():
        wf_ref[...] = jnp.dot(wp_ref[...], w1_ref[...],
                              preferred_element_type=jnp.float32)
        bf_ref[...] = jnp.dot(bp_ref[...], w1_ref[...],
                              preferred_element_type=jnp.float32) + b1_ref[...]
    xg = x_ref[...] * g_ref[...]
    lin = jnp.dot(xg, wf_ref[...], preferred_element_type=jnp.float32)
    out_ref[...] = (lin + bf_ref[...]) * rs_ref[...]


def _tc_matmul(x, g, Wp, W1, bp, b1, rs_col):
    grid = N // _BLK
    return pl.pallas_call(
        _tc_matmul_body,
        grid=(grid,),
        in_specs=[
            pl.BlockSpec((_BLK, D), lambda i: (i, 0)),
            pl.BlockSpec((1, D), lambda i: (0, 0)),
            pl.BlockSpec((D, H), lambda i: (0, 0)),
            pl.BlockSpec((H, H), lambda i: (0, 0)),
            pl.BlockSpec((1, H), lambda i: (0, 0)),
            pl.BlockSpec((1, H), lambda i: (0, 0)),
            pl.BlockSpec((_BLK, 1), lambda i: (i, 0)),
        ],
        out_specs=pl.BlockSpec((_BLK, H), lambda i: (i, 0)),
        out_shape=jax.ShapeDtypeStruct((N, H), jnp.float32),
        scratch_shapes=[
            pltpu.VMEM((D, H), jnp.float32),
            pltpu.VMEM((1, H), jnp.float32),
        ],
    )(x, g, Wp, W1, bp, b1, rs_col)


def _tc_final_body(s0_ref, s1_ref, tab_ref, rs_ref, cw_ref, w2_ref, b2_ref,
                   out_ref, t_acc, h1r0, scal):
    i = pl.program_id(0)
    n = pl.num_programs(0)
    h1 = jnp.maximum(rs_ref[...] * (s0_ref[...] + s1_ref[...] + tab_ref[...]),
                     0.0)
    part = jnp.sum(cw_ref[...] * h1, axis=0, keepdims=True)
    ns = jnp.sum(cw_ref[...])

    @pl.when(i == 0)
    def _():
        t_acc[...] = part
        h1r0[...] = h1[0:1, :]
        scal[0] = ns
        scal[1] = rs_ref[0, 0]

    @pl.when(i > 0)
    def _():
        t_acc[...] = t_acc[...] + part
        scal[0] = scal[0] + ns

    @pl.when(i == n - 1)
    def _():
        rs0 = scal[1]
        nsum = scal[0]
        lin2_self = jnp.dot(h1r0[...], w2_ref[...],
                            preferred_element_type=jnp.float32) + b2_ref[...]
        agg2 = (rs0 * jnp.dot(t_acc[...], w2_ref[...],
                              preferred_element_type=jnp.float32)
                + (rs0 * nsum) * b2_ref[...]
                + lin2_self * (rs0 * rs0))
        out_ref[0:1, :] = h1r0[...]
        out_ref[1:2, :] = jnp.maximum(agg2, 0.0)


def _tc_final(s0, s1, tab, rs_col, cw_col, W2, b2):
    grid = N // _BLK
    return pl.pallas_call(
        _tc_final_body,
        grid=(grid,),
        in_specs=[
            pl.BlockSpec((_BLK, H), lambda i: (i, 0)),
            pl.BlockSpec((_BLK, H), lambda i: (i, 0)),
            pl.BlockSpec((_BLK, H), lambda i: (i, 0)),
            pl.BlockSpec((_BLK, 1), lambda i: (i, 0)),
            pl.BlockSpec((_BLK, 1), lambda i: (i, 0)),
            pl.BlockSpec((H, H), lambda i: (0, 0)),
            pl.BlockSpec((1, H), lambda i: (0, 0)),
        ],
        out_specs=pl.BlockSpec((2, H), lambda i: (0, 0)),
        out_shape=jax.ShapeDtypeStruct((2, H), jnp.float32),
        scratch_shapes=[
            pltpu.VMEM((1, H), jnp.float32),
            pltpu.VMEM((1, H), jnp.float32),
            pltpu.SMEM((2,), jnp.float32),
        ],
    )(s0, s1, tab, rs_col, cw_col, W2, b2)


@jax.jit
def kernel(x, edge_index, feat_gate, edge_gate, W_proj, b_proj, W1, b1, W2,
           b2):
    src = edge_index[0]
    dst = edge_index[1]
    pad = E_PAD - E
    if pad:
        zi = jnp.zeros((pad,), jnp.int32)
        src = jnp.concatenate([src, zi])
        dst = jnp.concatenate([dst, zi])
        w = jnp.concatenate([edge_gate, jnp.zeros((pad,), jnp.float32)])
    else:
        w = edge_gate

    deg_p, craw_p = _sc_edge_scalars(src, dst, w)
    rs, cw = _tc_scalars(deg_p, craw_p)
    rs_col = rs.reshape(N_PAD, 1)[:N]
    cw_col = cw.reshape(N_PAD, 1)[:N]

    tab = _tc_matmul(x, feat_gate.reshape(1, D), W_proj, W1,
                     b_proj.reshape(1, H), b1.reshape(1, H), rs_col)

    s_p = _sc_scatter_rows(src, dst, w, tab)
    out2 = _tc_final(s_p[0, :N], s_p[1, :N], tab, rs_col, cw_col, W2,
                     b2.reshape(1, H))
    return out2.reshape(2 * H)
